# baseline (speedup 1.0000x reference)
import functools

import jax
import jax.numpy as jnp
from jax import lax
from jax.experimental import pallas as pl
from jax.experimental.pallas import tpu as pltpu
from jax.experimental.pallas import tpu_sc as plsc

_ENT_DIM = 32
_REL_DIM = 32
_B = 16384
_MARGIN = 6.0
_L = 16
_NC = 2
_NS = 16
_NW = _NC * _NS
_BPW = _B // _NW
_G = _BPW // _L
_NBUF = 2


def _rsqrt_nt(x):
    i = plsc.bitcast(x, jnp.int32)
    i = jnp.int32(0x5F3759DF) - lax.shift_right_logical(i, 1)
    y = plsc.bitcast(i, jnp.float32)
    for _ in range(3):
        y = y * (1.5 - ((0.5 * x) * y) * y)
    return y


def _make_sc_kernel():
    mesh = plsc.VectorSubcoreMesh(core_axis_name="c", subcore_axis_name="s")

    buf_types = []
    for _ in range(_NBUF):
        for _ in range(2):
            buf_types += [
                pltpu.VMEM((_L, _ENT_DIM), jnp.float32),
                pltpu.VMEM((_L, _ENT_DIM), jnp.float32),
                pltpu.VMEM((_L, _REL_DIM), jnp.float32),
                pltpu.VMEM((_L, _REL_DIM * _ENT_DIM), jnp.float32),
            ]

    @functools.partial(
        pl.kernel,
        out_type=jax.ShapeDtypeStruct((_NW, _L), jnp.float32),
        mesh=mesh,
        compiler_params=pltpu.CompilerParams(needs_layout_passes=False,
                                             use_tc_tiling_on_sc=False),
        scratch_types=[
            pltpu.VMEM((6 * _BPW,), jnp.int32),
            pltpu.VMEM((_ENT_DIM, _L), jnp.float32),
            pltpu.VMEM((_L,), jnp.float32),
        ] + buf_types + [
            pltpu.SemaphoreType.DMA,
            pltpu.SemaphoreType.DMA,
        ],
    )
    def sc_loss(idx_hbm, ent_hbm, rel_hbm, proj_hbm, out_hbm,
                idx_v, xs_v, acc_v, *bufs_and_sems):
        refs = bufs_and_sems[:-2]
        sems = bufs_and_sems[-2:]
        bufs = tuple(
            tuple(tuple(refs[(b * 2 + s) * 4:(b * 2 + s) * 4 + 4])
                  for s in range(2))
            for b in range(_NBUF))

        wid = lax.axis_index("s") * _NC + lax.axis_index("c")
        base = wid * _BPW
        lane = lax.iota(jnp.int32, _L)
        cols = [jnp.full((_L,), j, jnp.int32) for j in range(_ENT_DIM)]
        zero = jnp.zeros((_L,), jnp.float32)

        for k in range(6):
            pltpu.sync_copy(idx_hbm.at[k, pl.ds(base, _BPW)],
                            idx_v.at[pl.ds(k * _BPW, _BPW)])

        def issue(g, b):
            off = g * _L
            for s in range(2):
                so = off + s * (3 * _BPW)
                hv = idx_v[pl.ds(so, _L)]
                rv = idx_v[pl.ds(so + _BPW, _L)]
                tv = idx_v[pl.ds(so + 2 * _BPW, _L)]
                eh_v, et_v, r_v, proj_v = bufs[b][s]
                pltpu.async_copy(ent_hbm.at[hv], eh_v, sems[b])
                pltpu.async_copy(ent_hbm.at[tv], et_v, sems[b])
                pltpu.async_copy(rel_hbm.at[rv], r_v, sems[b])
                pltpu.async_copy(proj_hbm.at[rv], proj_v, sems[b])

        def drain(b):
            iv = idx_v[pl.ds(0, _L)]
            for s in range(2):
                eh_v, et_v, r_v, proj_v = bufs[b][s]
                pltpu.make_async_copy(ent_hbm.at[iv], eh_v, sems[b]).wait()
                pltpu.make_async_copy(ent_hbm.at[iv], et_v, sems[b]).wait()
                pltpu.make_async_copy(rel_hbm.at[iv], r_v, sems[b]).wait()
                pltpu.make_async_copy(proj_hbm.at[iv], proj_v, sems[b]).wait()

        def distance(side):
            eh_v, et_v, r_v, proj_v = side

            def gth(ref, col):
                return plsc.load_gather(ref, [lane, col])

            sh0 = sh1 = st0 = st1 = zero
            for j in range(0, _ENT_DIM, 2):
                h0 = gth(eh_v, cols[j])
                t0 = gth(et_v, cols[j])
                h1 = gth(eh_v, cols[j + 1])
                t1 = gth(et_v, cols[j + 1])
                sh0 = sh0 + h0 * h0
                st0 = st0 + t0 * t0
                sh1 = sh1 + h1 * h1
                st1 = st1 + t1 * t1
            ih = _rsqrt_nt(sh0 + sh1)
            it = _rsqrt_nt(st0 + st1)
            for j in range(_ENT_DIM):
                xs_v[j, :] = gth(eh_v, cols[j]) * ih - gth(et_v, cols[j]) * it

            def inner(i4, dsq):
                i = i4 * 4
                ib = [jnp.full((_L,), (i + k) * _ENT_DIM, jnp.int32)
                      for k in range(4)]
                a = [zero, zero, zero, zero]
                for j in range(_ENT_DIM):
                    xj = xs_v[j, :]
                    for k in range(4):
                        a[k] = a[k] + gth(proj_v, ib[k] + cols[j]) * xj
                for k in range(4):
                    v = a[k] + gth(r_v, jnp.full((_L,), i + k, jnp.int32))
                    dsq = dsq + v * v
                return dsq

            dsq = lax.fori_loop(0, _REL_DIM // 4, inner, zero)
            return dsq * _rsqrt_nt(dsq)

        issue(0, 0)
        issue(1, 1)

        def pair(gg, acc):
            for b in range(_NBUF):
                g = gg * _NBUF + b
                drain(b)
                dp = distance(bufs[b][0])
                dn = distance(bufs[b][1])
                acc = acc + jnp.maximum(dp - dn + _MARGIN, 0.0)
                issue(jnp.minimum(g + _NBUF, _G - 1), b)
            return acc

        acc = lax.fori_loop(0, _G // _NBUF, pair, zero)
        drain(0)
        drain(1)
        acc_v[...] = acc
        pltpu.sync_copy(acc_v, out_hbm.at[wid])

    return sc_loss


_SC_LOSS = _make_sc_kernel()


@jax.jit
def kernel(pos_triples, neg_triples, ent_emb, rel_emb, proj_matrix):
    idx = jnp.concatenate([pos_triples, neg_triples], axis=0).astype(jnp.int32)
    partials = _SC_LOSS(idx, ent_emb, rel_emb, proj_matrix)
    return jnp.sum(partials) / jnp.float32(_B)


# speedup vs baseline: 1.3231x; 1.3231x over previous
import functools

import jax
import jax.numpy as jnp
from jax import lax
from jax.experimental import pallas as pl
from jax.experimental.pallas import tpu as pltpu
from jax.experimental.pallas import tpu_sc as plsc

_ENT_DIM = 32
_REL_DIM = 32
_NROW = 100000
_B = 16384
_MARGIN = 6.0
_L = 16
_NC = 2
_NS = 16
_NW = _NC * _NS
_BPW = _B // _NW
_G = _BPW // _L
_NBUF = 2
_CH = 8


def _rsqrt_nt(x):
    i = plsc.bitcast(x, jnp.int32)
    i = jnp.int32(0x5F3759DF) - lax.shift_right_logical(i, 1)
    y = plsc.bitcast(i, jnp.float32)
    for _ in range(3):
        y = y * (1.5 - ((0.5 * x) * y) * y)
    return y


def _make_sc_kernel():
    mesh = plsc.VectorSubcoreMesh(core_axis_name="c", subcore_axis_name="s")

    buf_types = []
    for _ in range(_NBUF):
        for _ in range(2):
            buf_types += [
                pltpu.VMEM((_L, _ENT_DIM), jnp.float32),
                pltpu.VMEM((_L, _ENT_DIM), jnp.float32),
                pltpu.VMEM((_L, _REL_DIM), jnp.float32),
                pltpu.VMEM((_CH, _L, 128), jnp.float32),
            ]

    @functools.partial(
        pl.kernel,
        out_type=jax.ShapeDtypeStruct((_NW, _L), jnp.float32),
        mesh=mesh,
        compiler_params=pltpu.CompilerParams(needs_layout_passes=False,
                                             use_tc_tiling_on_sc=False),
        scratch_types=[
            pltpu.VMEM((6 * _BPW,), jnp.int32),
            pltpu.VMEM((_ENT_DIM, _L), jnp.float32),
            pltpu.VMEM((_L,), jnp.float32),
        ] + buf_types + [
            pltpu.SemaphoreType.DMA,
            pltpu.SemaphoreType.DMA,
        ],
    )
    def sc_loss(idx_hbm, ent_hbm, rel_hbm, proj_hbm, out_hbm,
                idx_v, xs_v, acc_v, *bufs_and_sems):
        refs = bufs_and_sems[:-2]
        sems = bufs_and_sems[-2:]
        bufs = tuple(
            tuple(tuple(refs[(b * 2 + s) * 4:(b * 2 + s) * 4 + 4])
                  for s in range(2))
            for b in range(_NBUF))

        wid = lax.axis_index("s") * _NC + lax.axis_index("c")
        base = wid * _BPW
        lane = lax.iota(jnp.int32, _L)
        cols = [jnp.full((_L,), j, jnp.int32) for j in range(_ENT_DIM)]
        zero = jnp.zeros((_L,), jnp.float32)

        for k in range(6):
            pltpu.sync_copy(idx_hbm.at[k, pl.ds(base, _BPW)],
                            idx_v.at[pl.ds(k * _BPW, _BPW)])

        def issue(g, b):
            off = g * _L
            for s in range(2):
                so = off + s * (3 * _BPW)
                hv = idx_v[pl.ds(so, _L)]
                rv = idx_v[pl.ds(so + _BPW, _L)]
                tv = idx_v[pl.ds(so + 2 * _BPW, _L)]
                eh_v, et_v, r_v, proj_v = bufs[b][s]
                pltpu.async_copy(ent_hbm.at[hv], eh_v, sems[b])
                pltpu.async_copy(ent_hbm.at[tv], et_v, sems[b])
                pltpu.async_copy(rel_hbm.at[rv], r_v, sems[b])
                cix = (lax.shift_left(lax.shift_right_logical(rv, 3), 6)
                       | lax.bitwise_and(rv, 7))
                for c8 in range(_CH):
                    pltpu.async_copy(proj_hbm.at[cix + (c8 * 8)],
                                     proj_v.at[c8], sems[b])

        def drain(b):
            iv = idx_v[pl.ds(0, _L)]
            for s in range(2):
                eh_v, et_v, r_v, proj_v = bufs[b][s]
                pltpu.make_async_copy(ent_hbm.at[iv], eh_v, sems[b]).wait()
                pltpu.make_async_copy(ent_hbm.at[iv], et_v, sems[b]).wait()
                pltpu.make_async_copy(rel_hbm.at[iv], r_v, sems[b]).wait()
                for c8 in range(_CH):
                    pltpu.make_async_copy(proj_hbm.at[iv], proj_v.at[c8],
                                          sems[b]).wait()

        def distance(side):
            eh_v, et_v, r_v, proj_v = side

            def gth(ref, col):
                return plsc.load_gather(ref, [lane, col])

            sh0 = sh1 = st0 = st1 = zero
            for j in range(0, _ENT_DIM, 2):
                h0 = gth(eh_v, cols[j])
                t0 = gth(et_v, cols[j])
                h1 = gth(eh_v, cols[j + 1])
                t1 = gth(et_v, cols[j + 1])
                sh0 = sh0 + h0 * h0
                st0 = st0 + t0 * t0
                sh1 = sh1 + h1 * h1
                st1 = st1 + t1 * t1
            ih = _rsqrt_nt(sh0 + sh1)
            it = _rsqrt_nt(st0 + st1)
            for j in range(_ENT_DIM):
                xs_v[j, :] = gth(eh_v, cols[j]) * ih - gth(et_v, cols[j]) * it

            def inner(i4, dsq):
                ch = jnp.full((_L,), i4, jnp.int32)
                a = [zero, zero, zero, zero]
                for j in range(_ENT_DIM):
                    xj = xs_v[j, :]
                    for k in range(4):
                        pj = plsc.load_gather(
                            proj_v, [ch, lane, cols[j] + (32 * k)])
                        a[k] = a[k] + pj * xj
                for k in range(4):
                    ri = gth(r_v, jnp.full((_L,), i4 * 4 + k, jnp.int32))
                    v = a[k] + ri
                    dsq = dsq + v * v
                return dsq

            dsq = lax.fori_loop(0, _REL_DIM // 4, inner, zero)
            return dsq * _rsqrt_nt(dsq)

        issue(0, 0)
        issue(1, 1)

        def pair(gg, acc):
            for b in range(_NBUF):
                g = gg * _NBUF + b
                drain(b)
                dp = distance(bufs[b][0])
                dn = distance(bufs[b][1])
                acc = acc + jnp.maximum(dp - dn + _MARGIN, 0.0)
                issue(jnp.minimum(g + _NBUF, _G - 1), b)
            return acc

        acc = lax.fori_loop(0, _G // _NBUF, pair, zero)
        drain(0)
        drain(1)
        acc_v[...] = acc
        pltpu.sync_copy(acc_v, out_hbm.at[wid])

    return sc_loss


_SC_LOSS = _make_sc_kernel()


@jax.jit
def kernel(pos_triples, neg_triples, ent_emb, rel_emb, proj_matrix):
    idx = jnp.concatenate([pos_triples, neg_triples], axis=0).astype(jnp.int32)
    pview = (proj_matrix.reshape(_NROW // 8, 8, _CH, 128)
             .transpose(0, 2, 1, 3)
             .reshape(_NROW * _CH, 128))
    partials = _SC_LOSS(idx, ent_emb, rel_emb, pview)
    return jnp.sum(partials) / jnp.float32(_B)


# speedup vs baseline: 2.6365x; 1.9926x over previous
import functools

import jax
import jax.numpy as jnp
from jax import lax
from jax.experimental import pallas as pl
from jax.experimental.pallas import tpu as pltpu
from jax.experimental.pallas import tpu_sc as plsc

_ENT_DIM = 32
_REL_DIM = 32
_NROW = 100000
_B = 16384
_MARGIN = 6.0
_L = 16
_NC = 2
_NS = 16
_NW = _NC * _NS
_BPW = _B // _NW
_G = _BPW // _L
_NBUF = 2
_CH = 8


def _rsqrt_nt(x):
    i = plsc.bitcast(x, jnp.int32)
    i = jnp.int32(0x5F3759DF) - lax.shift_right_logical(i, 1)
    y = plsc.bitcast(i, jnp.float32)
    for _ in range(3):
        y = y * (1.5 - ((0.5 * x) * y) * y)
    return y


def _make_sc_kernel():
    mesh = plsc.VectorSubcoreMesh(core_axis_name="c", subcore_axis_name="s")

    buf_types = []
    for _ in range(_NBUF):
        for _ in range(2):
            buf_types += [
                pltpu.VMEM((_L, _ENT_DIM), jnp.float32),
                pltpu.VMEM((_L, _ENT_DIM), jnp.float32),
                pltpu.VMEM((_L, _REL_DIM), jnp.float32),
                pltpu.VMEM((_CH, _L, 128), jnp.float32),
            ]

    @functools.partial(
        pl.kernel,
        out_type=jax.ShapeDtypeStruct((_NW, _L), jnp.float32),
        mesh=mesh,
        compiler_params=pltpu.CompilerParams(needs_layout_passes=False,
                                             use_tc_tiling_on_sc=False),
        scratch_types=[
            pltpu.VMEM((6 * _BPW,), jnp.int32),
            pltpu.VMEM((_ENT_DIM, _L), jnp.float32),
            pltpu.VMEM((_L,), jnp.float32),
        ] + buf_types + [
            pltpu.SemaphoreType.DMA,
            pltpu.SemaphoreType.DMA,
        ],
    )
    def sc_loss(idx_hbm, ent_hbm, rel_hbm, proj_hbm, out_hbm,
                idx_v, xs_v, acc_v, *bufs_and_sems):
        refs = bufs_and_sems[:-2]
        sems = bufs_and_sems[-2:]
        bufs = tuple(
            tuple(tuple(refs[(b * 2 + s) * 4:(b * 2 + s) * 4 + 4])
                  for s in range(2))
            for b in range(_NBUF))

        wid = lax.axis_index("s") * _NC + lax.axis_index("c")
        base = wid * _BPW
        lane = lax.iota(jnp.int32, _L)
        rot = [lax.bitwise_and(lane + j, _ENT_DIM - 1) for j in range(_ENT_DIM)]
        zero = jnp.zeros((_L,), jnp.float32)

        for k in range(6):
            pltpu.sync_copy(idx_hbm.at[k, pl.ds(base, _BPW)],
                            idx_v.at[pl.ds(k * _BPW, _BPW)])

        def issue(g, b):
            off = g * _L
            for s in range(2):
                so = off + s * (3 * _BPW)
                hv = idx_v[pl.ds(so, _L)]
                rv = idx_v[pl.ds(so + _BPW, _L)]
                tv = idx_v[pl.ds(so + 2 * _BPW, _L)]
                eh_v, et_v, r_v, proj_v = bufs[b][s]
                pltpu.async_copy(ent_hbm.at[hv], eh_v, sems[b])
                pltpu.async_copy(ent_hbm.at[tv], et_v, sems[b])
                pltpu.async_copy(rel_hbm.at[rv], r_v, sems[b])
                cix = (lax.shift_left(lax.shift_right_logical(rv, 3), 6)
                       | lax.bitwise_and(rv, 7))
                for c8 in range(_CH):
                    pltpu.async_copy(proj_hbm.at[cix + (c8 * 8)],
                                     proj_v.at[c8], sems[b])

        def drain(b):
            iv = idx_v[pl.ds(0, _L)]
            for s in range(2):
                eh_v, et_v, r_v, proj_v = bufs[b][s]
                pltpu.make_async_copy(ent_hbm.at[iv], eh_v, sems[b]).wait()
                pltpu.make_async_copy(ent_hbm.at[iv], et_v, sems[b]).wait()
                pltpu.make_async_copy(rel_hbm.at[iv], r_v, sems[b]).wait()
                for c8 in range(_CH):
                    pltpu.make_async_copy(proj_hbm.at[iv], proj_v.at[c8],
                                          sems[b]).wait()

        def distance(side):
            eh_v, et_v, r_v, proj_v = side

            def gth(ref, col):
                return plsc.load_gather(ref, [lane, col])

            sh0 = sh1 = st0 = st1 = zero
            for j in range(0, _ENT_DIM, 2):
                h0 = gth(eh_v, rot[j])
                t0 = gth(et_v, rot[j])
                h1 = gth(eh_v, rot[j + 1])
                t1 = gth(et_v, rot[j + 1])
                sh0 = sh0 + h0 * h0
                st0 = st0 + t0 * t0
                sh1 = sh1 + h1 * h1
                st1 = st1 + t1 * t1
            ih = _rsqrt_nt(sh0 + sh1)
            it = _rsqrt_nt(st0 + st1)
            for j in range(_ENT_DIM):
                xs_v[j, :] = gth(eh_v, rot[j]) * ih - gth(et_v, rot[j]) * it

            def inner(i4, dsq):
                ch = jnp.full((_L,), i4, jnp.int32)
                a = [zero, zero, zero, zero]
                for j in range(_ENT_DIM):
                    xj = xs_v[j, :]
                    for k in range(4):
                        pj = plsc.load_gather(
                            proj_v, [ch, lane, rot[j] + (32 * k)])
                        a[k] = a[k] + pj * xj
                for k in range(4):
                    ri = gth(r_v, jnp.full((_L,), i4 * 4 + k, jnp.int32))
                    v = a[k] + ri
                    dsq = dsq + v * v
                return dsq

            dsq = lax.fori_loop(0, _REL_DIM // 4, inner, zero)
            return dsq * _rsqrt_nt(dsq)

        issue(0, 0)
        issue(1, 1)

        def pair(gg, acc):
            for b in range(_NBUF):
                g = gg * _NBUF + b
                drain(b)
                dp = distance(bufs[b][0])
                dn = distance(bufs[b][1])
                acc = acc + jnp.maximum(dp - dn + _MARGIN, 0.0)
                issue(jnp.minimum(g + _NBUF, _G - 1), b)
            return acc

        acc = lax.fori_loop(0, _G // _NBUF, pair, zero)
        drain(0)
        drain(1)
        acc_v[...] = acc
        pltpu.sync_copy(acc_v, out_hbm.at[wid])

    return sc_loss


_SC_LOSS = _make_sc_kernel()


@jax.jit
def kernel(pos_triples, neg_triples, ent_emb, rel_emb, proj_matrix):
    idx = jnp.concatenate([pos_triples, neg_triples], axis=0).astype(jnp.int32)
    pview = (proj_matrix.reshape(_NROW // 8, 8, _CH, 128)
             .transpose(0, 2, 1, 3)
             .reshape(_NROW * _CH, 128))
    partials = _SC_LOSS(idx, ent_emb, rel_emb, pview)
    return jnp.sum(partials) / jnp.float32(_B)


# speedup vs baseline: 2.6384x; 1.0007x over previous
import functools

import jax
import jax.numpy as jnp
from jax import lax
from jax.experimental import pallas as pl
from jax.experimental.pallas import tpu as pltpu
from jax.experimental.pallas import tpu_sc as plsc

_ENT_DIM = 32
_REL_DIM = 32
_NROW = 100000
_B = 16384
_MARGIN = 6.0
_L = 16
_NC = 2
_NS = 16
_NW = _NC * _NS
_BPW = _B // _NW
_G = _BPW // _L
_NBUF = 2
_CH = 8


def _rsqrt_nt(x):
    i = plsc.bitcast(x, jnp.int32)
    i = jnp.int32(0x5F3759DF) - lax.shift_right_logical(i, 1)
    y = plsc.bitcast(i, jnp.float32)
    for _ in range(3):
        y = y * (1.5 - ((0.5 * x) * y) * y)
    return y


def _make_sc_kernel():
    mesh = plsc.VectorSubcoreMesh(core_axis_name="c", subcore_axis_name="s")

    buf_types = []
    for _ in range(_NBUF):
        for _ in range(2):
            buf_types += [
                pltpu.VMEM((_L, _ENT_DIM), jnp.float32),
                pltpu.VMEM((_L, _ENT_DIM), jnp.float32),
                pltpu.VMEM((_L, _REL_DIM), jnp.float32),
                pltpu.VMEM((_CH, _L, 128), jnp.float32),
            ]

    @functools.partial(
        pl.kernel,
        out_type=jax.ShapeDtypeStruct((_NW, _L), jnp.float32),
        mesh=mesh,
        compiler_params=pltpu.CompilerParams(needs_layout_passes=False,
                                             use_tc_tiling_on_sc=False),
        scratch_types=[
            pltpu.VMEM((6 * _BPW,), jnp.int32),
            pltpu.VMEM((2 * _ENT_DIM, _L), jnp.float32),
            pltpu.VMEM((_L,), jnp.float32),
        ] + buf_types + [
            pltpu.SemaphoreType.DMA,
            pltpu.SemaphoreType.DMA,
        ],
    )
    def sc_loss(idx_hbm, ent_hbm, rel_hbm, proj_hbm, out_hbm,
                idx_v, xs_v, acc_v, *bufs_and_sems):
        refs = bufs_and_sems[:-2]
        sems = bufs_and_sems[-2:]
        bufs = tuple(
            tuple(tuple(refs[(b * 2 + s) * 4:(b * 2 + s) * 4 + 4])
                  for s in range(2))
            for b in range(_NBUF))

        wid = lax.axis_index("s") * _NC + lax.axis_index("c")
        base = wid * _BPW
        lane = lax.iota(jnp.int32, _L)
        rot = [lax.bitwise_and(lane + j, _ENT_DIM - 1) for j in range(_ENT_DIM)]
        zero = jnp.zeros((_L,), jnp.float32)

        for k in range(6):
            pltpu.sync_copy(idx_hbm.at[k, pl.ds(base, _BPW)],
                            idx_v.at[pl.ds(k * _BPW, _BPW)])

        def issue(g, b):
            off = g * _L
            for s in range(2):
                so = off + s * (3 * _BPW)
                hv = idx_v[pl.ds(so, _L)]
                rv = idx_v[pl.ds(so + _BPW, _L)]
                tv = idx_v[pl.ds(so + 2 * _BPW, _L)]
                eh_v, et_v, r_v, proj_v = bufs[b][s]
                pltpu.async_copy(ent_hbm.at[hv], eh_v, sems[b])
                pltpu.async_copy(ent_hbm.at[tv], et_v, sems[b])
                pltpu.async_copy(rel_hbm.at[rv], r_v, sems[b])
                cix = (lax.shift_left(lax.shift_right_logical(rv, 3), 6)
                       | lax.bitwise_and(rv, 7))
                for c8 in range(_CH):
                    pltpu.async_copy(proj_hbm.at[cix + (c8 * 8)],
                                     proj_v.at[c8], sems[b])

        def drain(b):
            iv = idx_v[pl.ds(0, _L)]
            for s in range(2):
                eh_v, et_v, r_v, proj_v = bufs[b][s]
                pltpu.make_async_copy(ent_hbm.at[iv], eh_v, sems[b]).wait()
                pltpu.make_async_copy(ent_hbm.at[iv], et_v, sems[b]).wait()
                pltpu.make_async_copy(rel_hbm.at[iv], r_v, sems[b]).wait()
                for c8 in range(_CH):
                    pltpu.make_async_copy(proj_hbm.at[iv], proj_v.at[c8],
                                          sems[b]).wait()

        def distance(side, xo):
            eh_v, et_v, r_v, proj_v = side

            def gth(ref, col):
                return plsc.load_gather(ref, [lane, col])

            sh0 = sh1 = st0 = st1 = zero
            for j in range(0, _ENT_DIM, 2):
                h0 = gth(eh_v, rot[j])
                t0 = gth(et_v, rot[j])
                h1 = gth(eh_v, rot[j + 1])
                t1 = gth(et_v, rot[j + 1])
                sh0 = sh0 + h0 * h0
                st0 = st0 + t0 * t0
                sh1 = sh1 + h1 * h1
                st1 = st1 + t1 * t1
            ih = _rsqrt_nt(sh0 + sh1)
            it = _rsqrt_nt(st0 + st1)
            for j in range(_ENT_DIM):
                xs_v[xo + j, :] = (gth(eh_v, rot[j]) * ih
                                   - gth(et_v, rot[j]) * it)

            def inner(i4, dsq):
                ch = jnp.full((_L,), i4, jnp.int32)
                a = [zero, zero, zero, zero]
                for j in range(_ENT_DIM):
                    xj = xs_v[xo + j, :]
                    for k in range(4):
                        pj = plsc.load_gather(
                            proj_v, [ch, lane, rot[j] + (32 * k)])
                        a[k] = a[k] + pj * xj
                for k in range(4):
                    ri = gth(r_v, jnp.full((_L,), i4 * 4 + k, jnp.int32))
                    v = a[k] + ri
                    dsq = dsq + v * v
                return dsq

            dsq = lax.fori_loop(0, _REL_DIM // 4, inner, zero)
            return dsq * _rsqrt_nt(dsq)

        issue(0, 0)
        issue(1, 1)

        def pair(gg, acc):
            for b in range(_NBUF):
                g = gg * _NBUF + b
                drain(b)
                dp = distance(bufs[b][0], 0)
                dn = distance(bufs[b][1], _ENT_DIM)
                acc = acc + jnp.maximum(dp - dn + _MARGIN, 0.0)
                issue(jnp.minimum(g + _NBUF, _G - 1), b)
            return acc

        acc = lax.fori_loop(0, _G // _NBUF, pair, zero)
        drain(0)
        drain(1)
        acc_v[...] = acc
        pltpu.sync_copy(acc_v, out_hbm.at[wid])

    return sc_loss


_SC_LOSS = _make_sc_kernel()


@jax.jit
def kernel(pos_triples, neg_triples, ent_emb, rel_emb, proj_matrix):
    idx = jnp.concatenate([pos_triples, neg_triples], axis=0).astype(jnp.int32)
    pview = (proj_matrix.reshape(_NROW // 8, 8, _CH, 128)
             .transpose(0, 2, 1, 3)
             .reshape(_NROW * _CH, 128))
    partials = _SC_LOSS(idx, ent_emb, rel_emb, pview)
    return jnp.sum(partials) / jnp.float32(_B)
